# Initial kernel scaffold; baseline (speedup 1.0000x reference)
import jax
import jax.numpy as jnp
from jax.experimental import pallas as pl


def kernel(x, table, W1, b1, gamma, beta, W2, b2):
    raise NotImplementedError("write your pallas kernel here")


# speedup vs baseline: 3.3102x; 3.3102x over previous
import functools

import jax
import jax.numpy as jnp
from jax import lax
from jax.experimental import pallas as pl
from jax.experimental.pallas import tpu as pltpu
from jax.experimental.pallas import tpu_sc as plsc

_VOCAB = 1000000
_HID = 64
_B = 16384
_L = 200
_EPS = 1e-5

_NC = 2
_NS = 16
_NW = _NC * _NS
_PER_W = _B // _NW
_HALF = _L // 2
_CHUNK = 64
_NCHUNK = _PER_W // _CHUNK
_NBUF = 4


def _sc_body(x_hbm, tab_hbm, out_hbm, idx_v, rows_v, out_v,
             sem0, sem1, sem2, sem3):
    sems = (sem0, sem1, sem2, sem3)
    wid = lax.axis_index("s") * _NC + lax.axis_index("c")
    base = wid * _PER_W

    def start(e_loc, b):
        for h in range(2):
            pltpu.async_copy(
                tab_hbm.at[idx_v.at[e_loc, h]],
                rows_v.at[b, pl.ds(h * _HALF, _HALF)],
                sems[b],
            )

    def wait(e_loc, b):
        for h in range(2):
            pltpu.make_async_copy(
                tab_hbm.at[idx_v.at[e_loc, h]],
                rows_v.at[b, pl.ds(h * _HALF, _HALF)],
                sems[b],
            ).wait()

    def accumulate(b):
        rbuf = rows_v.at[b]

        def acc_body(j, carry):
            a0, a1, a2, a3 = carry
            a0 = a0 + rbuf[j, pl.ds(0, 16)]
            a1 = a1 + rbuf[j, pl.ds(16, 16)]
            a2 = a2 + rbuf[j, pl.ds(32, 16)]
            a3 = a3 + rbuf[j, pl.ds(48, 16)]
            return (a0, a1, a2, a3)

        zero = jnp.zeros((16,), jnp.float32)
        return lax.fori_loop(0, _L, acc_body, (zero, zero, zero, zero),
                             unroll=8)

    def chunk(ci, _):
        cb = base + ci * _CHUNK
        pltpu.sync_copy(x_hbm.at[pl.ds(cb, _CHUNK)], idx_v)
        for b in range(_NBUF):
            start(b, b)

        def group(t, _):
            for b in range(_NBUF):
                e = t * _NBUF + b
                wait(e, b)
                a0, a1, a2, a3 = accumulate(b)

                @pl.when(e + _NBUF < _CHUNK)
                def _():
                    start(e + _NBUF, b)

                row = ci * _CHUNK + e
                out_v[row, pl.ds(0, 16)] = a0
                out_v[row, pl.ds(16, 16)] = a1
                out_v[row, pl.ds(32, 16)] = a2
                out_v[row, pl.ds(48, 16)] = a3
            return 0

        lax.fori_loop(0, _CHUNK // _NBUF, group, 0)
        return 0

    lax.fori_loop(0, _NCHUNK, chunk, 0)
    pltpu.sync_copy(out_v, out_hbm.at[pl.ds(base, _PER_W)])


@functools.cache
def _sc_bow_sums():
    return pl.kernel(
        _sc_body,
        out_type=jax.ShapeDtypeStruct((_B, _HID), jnp.float32),
        mesh=plsc.VectorSubcoreMesh(core_axis_name="c", subcore_axis_name="s"),
        compiler_params=pltpu.CompilerParams(use_tc_tiling_on_sc=False),
        scratch_types=[
            pltpu.VMEM((_CHUNK, 2, _HALF), jnp.int32),
            pltpu.VMEM((_NBUF, _L, _HID), jnp.float32),
            pltpu.VMEM((_PER_W, _HID), jnp.float32),
            pltpu.SemaphoreType.DMA,
            pltpu.SemaphoreType.DMA,
            pltpu.SemaphoreType.DMA,
            pltpu.SemaphoreType.DMA,
        ],
    )


def _dense_body(sums_ref, w1_ref, b1_ref, g_ref, bt_ref, w2_ref, b2_ref,
                out_ref):
    bow = sums_ref[...] * (1.0 / _L)
    h = lax.dot_general(bow, w1_ref[...], (((1,), (1,)), ((), ())),
                        preferred_element_type=jnp.float32) + b1_ref[...]
    mu = jnp.mean(h, axis=0, keepdims=True)
    hc = h - mu
    var = jnp.mean(hc * hc, axis=0, keepdims=True)
    hn = hc * lax.rsqrt(var + _EPS) * g_ref[...] + bt_ref[...]
    h2 = jnp.maximum(hn, 0.0)
    out_ref[...] = lax.dot_general(h2, w2_ref[...], (((1,), (1,)), ((), ())),
                                   preferred_element_type=jnp.float32) + b2_ref[...]


def _dense_head(sums, w1, b1, g, bt, w2p, b2p):
    return pl.pallas_call(
        _dense_body,
        out_shape=jax.ShapeDtypeStruct((_B, 8), jnp.float32),
    )(sums, w1, b1, g, bt, w2p, b2p)


def kernel(x, table, W1, b1, gamma, beta, W2, b2):
    x3 = x.astype(jnp.int32).reshape(_B, 2, _HALF)
    sums = _sc_bow_sums()(x3, table)
    w2p = jnp.zeros((8, _HID), jnp.float32).at[:5, :].set(W2)
    b2p = jnp.zeros((1, 8), jnp.float32).at[:, :5].set(b2)
    out8 = _dense_head(sums, W1, b1[None, :], gamma[None, :], beta[None, :],
                       w2p, b2p)
    return out8[:, :5]


# speedup vs baseline: 3.4176x; 1.0325x over previous
import functools

import jax
import jax.numpy as jnp
from jax import lax
from jax.experimental import pallas as pl
from jax.experimental.pallas import tpu as pltpu
from jax.experimental.pallas import tpu_sc as plsc

_VOCAB = 1000000
_HID = 64
_B = 16384
_L = 200
_EPS = 1e-5

_NC = 2
_NS = 16
_NW = _NC * _NS
_PER_W = _B // _NW
_S0 = 104
_S1 = _L - _S0
_ICHUNK = 32
_NICHUNK = _PER_W // _ICHUNK
_OCHUNK = 64
_NBUF = 8


def _sc_body(x_hbm, tab_hbm, out_hbm, idx_v, rows_v, out_v,
             g0, g1, g2, g3, g4, g5, g6, g7, sem_idx, sem_out):
    gsems = (g0, g1, g2, g3, g4, g5, g6, g7)
    wid = lax.axis_index("s") * _NC + lax.axis_index("c")
    base = wid * _PER_W

    def idx_src(e):
        par = lax.rem(lax.div(e, _ICHUNK), 2)
        eloc = lax.rem(e, _ICHUNK)
        return par, eloc

    def start(e, b):
        par, eloc = idx_src(e)
        for off, n in ((0, _S0), (_S0, _S1)):
            pltpu.async_copy(
                tab_hbm.at[idx_v.at[par, eloc, pl.ds(off, n)]],
                rows_v.at[b, pl.ds(off, n)],
                gsems[b],
            )

    def wait_g(e, b):
        par, eloc = idx_src(e)
        for off, n in ((0, _S0), (_S0, _S1)):
            pltpu.make_async_copy(
                tab_hbm.at[idx_v.at[par, eloc, pl.ds(off, n)]],
                rows_v.at[b, pl.ds(off, n)],
                gsems[b],
            ).wait()

    def accumulate(b):
        rbuf = rows_v.at[b]

        def acc_body(j, carry):
            a0, a1, a2, a3 = carry
            a0 = a0 + rbuf[j, pl.ds(0, 16)]
            a1 = a1 + rbuf[j, pl.ds(16, 16)]
            a2 = a2 + rbuf[j, pl.ds(32, 16)]
            a3 = a3 + rbuf[j, pl.ds(48, 16)]
            return (a0, a1, a2, a3)

        zero = jnp.zeros((16,), jnp.float32)
        return lax.fori_loop(0, _L, acc_body, (zero, zero, zero, zero),
                             unroll=8)

    def drain_out():
        pltpu.make_async_copy(
            out_v.at[0], out_hbm.at[pl.ds(base, _OCHUNK)], sem_out,
        ).wait()

    pltpu.sync_copy(x_hbm.at[pl.ds(base, _ICHUNK)], idx_v.at[0])
    pltpu.async_copy(x_hbm.at[pl.ds(base + _ICHUNK, _ICHUNK)], idx_v.at[1],
                     sem_idx)
    for b in range(_NBUF):
        start(b, b)

    def group(t, _):
        for b in range(_NBUF):
            e = t * _NBUF + b
            wait_g(e, b)
            a0, a1, a2, a3 = accumulate(b)
            e2 = e + _NBUF

            @pl.when(jnp.logical_and(e2 < _PER_W, lax.rem(e2, _ICHUNK) == 0))
            def _():
                pltpu.make_async_copy(
                    x_hbm.at[pl.ds(base, _ICHUNK)], idx_v.at[0], sem_idx,
                ).wait()

            @pl.when(e2 < _PER_W)
            def _():
                start(e2, b)

            @pl.when(jnp.logical_and(
                jnp.logical_and(lax.rem(e, _ICHUNK) == 0, e > 0),
                e < (_NICHUNK - 1) * _ICHUNK))
            def _():
                cj1 = lax.div(e, _ICHUNK) + 1
                pltpu.async_copy(
                    x_hbm.at[pl.ds(base + cj1 * _ICHUNK, _ICHUNK)],
                    idx_v.at[lax.rem(cj1, 2)],
                    sem_idx,
                )

            opar = lax.rem(lax.div(e, _OCHUNK), 2)
            orow = lax.rem(e, _OCHUNK)
            inv_l = jnp.float32(1.0 / _L)
            out_v[opar, orow, pl.ds(0, 16)] = a0 * inv_l
            out_v[opar, orow, pl.ds(16, 16)] = a1 * inv_l
            out_v[opar, orow, pl.ds(32, 16)] = a2 * inv_l
            out_v[opar, orow, pl.ds(48, 16)] = a3 * inv_l

            @pl.when(lax.rem(e, _OCHUNK) == _OCHUNK - 1)
            def _():
                oc = lax.div(e, _OCHUNK)

                @pl.when(oc > 0)
                def _():
                    drain_out()

                pltpu.async_copy(
                    out_v.at[opar],
                    out_hbm.at[pl.ds(base + oc * _OCHUNK, _OCHUNK)],
                    sem_out,
                )
        return 0

    lax.fori_loop(0, _PER_W // _NBUF, group, 0)
    drain_out()


@functools.cache
def _sc_bow_sums():
    return pl.kernel(
        _sc_body,
        out_type=jax.ShapeDtypeStruct((_B, _HID), jnp.float32),
        mesh=plsc.VectorSubcoreMesh(core_axis_name="c", subcore_axis_name="s"),
        compiler_params=pltpu.CompilerParams(use_tc_tiling_on_sc=False),
        scratch_types=[
            pltpu.VMEM((2, _ICHUNK, _L), jnp.int32),
            pltpu.VMEM((_NBUF, _L, _HID), jnp.float32),
            pltpu.VMEM((2, _OCHUNK, _HID), jnp.float32),
            pltpu.SemaphoreType.DMA,
            pltpu.SemaphoreType.DMA,
            pltpu.SemaphoreType.DMA,
            pltpu.SemaphoreType.DMA,
            pltpu.SemaphoreType.DMA,
            pltpu.SemaphoreType.DMA,
            pltpu.SemaphoreType.DMA,
            pltpu.SemaphoreType.DMA,
            pltpu.SemaphoreType.DMA,
            pltpu.SemaphoreType.DMA,
        ],
    )


def _dense_body(sums_ref, w1_ref, b1_ref, g_ref, bt_ref, w2_ref, b2_ref,
                out_ref):
    bow = sums_ref[...]
    h = lax.dot_general(bow, w1_ref[...], (((1,), (1,)), ((), ())),
                        preferred_element_type=jnp.float32) + b1_ref[...]
    mu = jnp.mean(h, axis=0, keepdims=True)
    hc = h - mu
    var = jnp.mean(hc * hc, axis=0, keepdims=True)
    hn = hc * lax.rsqrt(var + _EPS) * g_ref[...] + bt_ref[...]
    h2 = jnp.maximum(hn, 0.0)
    out_ref[...] = lax.dot_general(h2, w2_ref[...], (((1,), (1,)), ((), ())),
                                   preferred_element_type=jnp.float32) + b2_ref[...]


def _dense_head(sums, w1, b1, g, bt, w2p, b2p):
    return pl.pallas_call(
        _dense_body,
        out_shape=jax.ShapeDtypeStruct((_B, 8), jnp.float32),
    )(sums, w1, b1, g, bt, w2p, b2p)


def kernel(x, table, W1, b1, gamma, beta, W2, b2):
    xi = x.astype(jnp.int32)
    bow = _sc_bow_sums()(xi, table)
    w2p = jnp.zeros((8, _HID), jnp.float32).at[:5, :].set(W2)
    b2p = jnp.zeros((1, 8), jnp.float32).at[:, :5].set(b2)
    out8 = _dense_head(bow, W1, b1[None, :], gamma[None, :], beta[None, :],
                       w2p, b2p)
    return out8[:, :5]
